# Initial kernel scaffold; baseline (speedup 1.0000x reference)
import jax
import jax.numpy as jnp
from jax.experimental import pallas as pl


def kernel(x):
    raise NotImplementedError("write your pallas kernel here")


# speedup vs baseline: 3.1239x; 3.1239x over previous
import functools

import jax
import jax.numpy as jnp
from jax.experimental import pallas as pl

H = W = 112
B = 64
G = 8

RATIOS = ((64, 64), (51, 79), (79, 51), (76, 53), (53, 76))
OUT_HW = tuple((H - kh + 1, W - kw + 1) for kh, kw in RATIOS)
SIZES = tuple(oh * ow for oh, ow in OUT_HW)
OFFSETS = (0, 2401, 4509, 6617, 8837)
BIG = 2**30


def _shift(a, k, axis):
    if k == 0:
        return a
    if axis == 0:
        pad = jnp.zeros((k, a.shape[1]), a.dtype)
        return jnp.concatenate([a[k:, :], pad], axis=0)
    pad = jnp.zeros((a.shape[0], k), a.dtype)
    return jnp.concatenate([a[:, k:], pad], axis=1)


def _pow2_sums(base, axis, max_pow):
    P = {1: base}
    p = 1
    while p < max_pow:
        P[2 * p] = P[p] + _shift(P[p], p, axis)
        p *= 2
    return P


def _window_sum(P, k, axis):
    acc = None
    off = 0
    for bit in (64, 32, 16, 8, 4, 2, 1):
        if k & bit:
            part = _shift(P[bit], off, axis) if off else P[bit]
            acc = part if acc is None else acc + part
            off += bit
    return acc


def _kernel_body(x_ref, s0, s1, s2, s3, s4, idx_ref, val_ref):
    xg = x_ref[...].reshape(G * H, W)
    PW = _pow2_sums(xg, axis=1, max_pow=64)
    outs = (s0, s1, s2, s3, s4)
    scores = []
    for r, (kh, kw) in enumerate(RATIOS):
        wsum = _window_sum(PW, kw, axis=1)
        PH = _pow2_sums(wsum, axis=0, max_pow=64)
        full = _window_sum(PH, kh, axis=0) * (1.0 / float(kh * kw))
        scores.append(full)
        oh, ow = OUT_HW[r]
        for b in range(G):
            outs[r][b, :, :] = full[b * H : b * H + oh, :ow]
    for b in range(G):
        best_val = None
        best_idx = None
        for r in range(4):
            oh, ow = OUT_HW[r]
            sub = scores[r][b * H : b * H + oh, :ow]
            m = jnp.max(sub)
            flat = (
                jax.lax.broadcasted_iota(jnp.int32, (oh, ow), 0) * ow
                + jax.lax.broadcasted_iota(jnp.int32, (oh, ow), 1)
                + OFFSETS[r]
            )
            cand = jnp.min(jnp.where(sub == m, flat, BIG))
            if best_val is None:
                best_val, best_idx = m, cand
            else:
                take_new = m > best_val
                best_idx = jnp.where(
                    take_new, cand, jnp.where(m == best_val, jnp.minimum(best_idx, cand), best_idx)
                )
                best_val = jnp.maximum(best_val, m)
        idx_ref[b : b + 1, 0:1] = best_idx[None, None]
        val_ref[b : b + 1, 0:1] = best_val[None, None]


@jax.jit
def _run(x3):
    grid = B // G
    out_shapes = [
        jax.ShapeDtypeStruct((B, oh, ow), jnp.float32) for oh, ow in OUT_HW
    ] + [
        jax.ShapeDtypeStruct((B, 1), jnp.int32),
        jax.ShapeDtypeStruct((B, 1), jnp.float32),
    ]
    grid_specs = [
        pl.BlockSpec((G, oh, ow), lambda i: (i, 0, 0)) for oh, ow in OUT_HW
    ] + [
        pl.BlockSpec((G, 1), lambda i: (i, 0)),
        pl.BlockSpec((G, 1), lambda i: (i, 0)),
    ]
    return pl.pallas_call(
        _kernel_body,
        grid=(grid,),
        in_specs=[pl.BlockSpec((G, H, W), lambda i: (i, 0, 0))],
        out_specs=grid_specs,
        out_shape=out_shapes,
    )(x3)


def kernel(x):
    x3 = x.reshape(B, H, W)
    *grids, idx, val = _run(x3)
    ws = jnp.concatenate([g.reshape(B, -1) for g in grids], axis=1)
    return (idx, val, ws)
